# baseline (speedup 1.0000x reference)
import functools

import jax
import jax.numpy as jnp
from jax import lax
from jax.experimental import pallas as pl
from jax.experimental.pallas import tpu as pltpu
from jax.experimental.pallas import tpu_sc as plsc

B = 16384
D = 32
F = 2 * D
NC = 2
NS = 16
NW = NC * NS
BPW = B // NW
CH = 128
NCH = BPW // CH


def _sc_gather_body(widx_u_hbm, widx_i_hbm, utab_hbm, itab_hbm,
                    x_out, idx_u, idx_i, xbuf, sem):
    wid = lax.axis_index("s") * NC + lax.axis_index("c")
    base = wid * BPW
    for j in range(NCH):
        pltpu.sync_copy(widx_u_hbm.at[pl.ds(base + j * CH, CH)], idx_u.at[j])
        pltpu.sync_copy(widx_i_hbm.at[pl.ds(base + j * CH, CH)], idx_i.at[j])

    def gather_feature(f, tab, idx, row):
        copies = [
            pltpu.async_copy(tab.at[f].at[idx.at[j]],
                             xbuf.at[row, pl.ds(j * CH, CH)], sem)
            for j in range(NCH)
        ]
        for c in copies:
            c.wait()

    def u_body(f, carry):
        gather_feature(f, utab_hbm, idx_u, f)
        return carry

    def i_body(f, carry):
        gather_feature(f, itab_hbm, idx_i, D + f)
        return carry

    lax.fori_loop(0, D, u_body, 0)
    lax.fori_loop(0, D, i_body, 0)
    pltpu.sync_copy(xbuf, x_out.at[:, pl.ds(base, BPW)])


_sc_gather = functools.partial(
    pl.kernel,
    out_type=jax.ShapeDtypeStruct((F, B), jnp.float32),
    mesh=plsc.VectorSubcoreMesh(core_axis_name="c", subcore_axis_name="s",
                                num_cores=NC, num_subcores=NS),
    scratch_types=[
        pltpu.VMEM((NCH, CH), jnp.int32),
        pltpu.VMEM((NCH, CH), jnp.int32),
        pltpu.VMEM((F, BPW), jnp.float32),
        pltpu.SemaphoreType.DMA,
    ],
    compiler_params=pltpu.CompilerParams(use_tc_tiling_on_sc=False),
)(_sc_gather_body)


BLK = 2048


def _mlp_body(x_ref, w1_ref, b1_ref, w2_ref, b2_ref, w3_ref, b3_ref, o_ref):
    h = jnp.dot(w1_ref[...], x_ref[...], preferred_element_type=jnp.float32)
    h = jnp.maximum(h + b1_ref[...], 0.0)
    h = jnp.dot(w2_ref[...], h, preferred_element_type=jnp.float32)
    h = jnp.maximum(h + b2_ref[...], 0.0)
    y = jnp.dot(w3_ref[...], h, preferred_element_type=jnp.float32)
    o_ref[...] = y[0] + b3_ref[0, 0]


def _mlp(xT, W1, b1c, W2, b2c, W3, b3c):
    grid = (B // BLK,)
    full = lambda shape: pl.BlockSpec(shape, lambda i: (0, 0))
    return pl.pallas_call(
        _mlp_body,
        grid=grid,
        in_specs=[
            pl.BlockSpec((F, BLK), lambda i: (0, i)),
            full((F, F)),
            full((F, 1)),
            full((D, F)),
            full((D, 1)),
            full((1, D)),
            full((1, 1)),
        ],
        out_specs=pl.BlockSpec((BLK,), lambda i: (i,)),
        out_shape=jax.ShapeDtypeStruct((B,), jnp.float32),
    )(xT, W1, b1c, W2, b2c, W3, b3c)


def kernel(user_idx, item_idx, user_table, item_table, W1, b1, W2, b2, W3, b3):
    ui = user_idx.astype(jnp.int32)
    ii = item_idx.astype(jnp.int32)
    xT = _sc_gather(ui, ii, user_table.T, item_table.T)
    out = _mlp(xT, W1, b1.reshape(F, 1), W2, b2.reshape(D, 1),
               W3, b3.reshape(1, 1))
    return out


# speedup vs baseline: 4.6707x; 4.6707x over previous
import functools

import jax
import jax.numpy as jnp
from jax import lax
from jax.experimental import pallas as pl
from jax.experimental.pallas import tpu as pltpu
from jax.experimental.pallas import tpu_sc as plsc

B = 16384
D = 32
F = 2 * D
NC = 2
NS = 16
NW = NC * NS
BPW = B // NW
CH = 128
NCH = BPW // CH
NV = 1000000 * D


def _sc_gather_body(gidx_hbm, flat_hbm, x_out, idxbuf, xbuf, sem):
    wid = lax.axis_index("s") * NC + lax.axis_index("c")
    base = wid * BPW
    pltpu.sync_copy(gidx_hbm.at[pl.ds(wid * (F * BPW), F * BPW)], idxbuf)

    def f_body(f, carry):
        for j in range(NCH):
            pltpu.async_copy(
                flat_hbm.at[idxbuf.at[pl.ds(f * BPW + j * CH, CH)]],
                xbuf.at[f, pl.ds(j * CH, CH)], sem)
        return carry

    lax.fori_loop(0, F, f_body, 0)
    pltpu.make_async_copy(x_out.at[:, pl.ds(base, BPW)], xbuf, sem).wait()
    pltpu.sync_copy(xbuf, x_out.at[:, pl.ds(base, BPW)])


_sc_gather = functools.partial(
    pl.kernel,
    out_type=jax.ShapeDtypeStruct((F, B), jnp.float32),
    mesh=plsc.VectorSubcoreMesh(core_axis_name="c", subcore_axis_name="s",
                                num_cores=NC, num_subcores=NS),
    scratch_types=[
        pltpu.VMEM((F * BPW,), jnp.int32),
        pltpu.VMEM((F, BPW), jnp.float32),
        pltpu.SemaphoreType.DMA,
    ],
    compiler_params=pltpu.CompilerParams(use_tc_tiling_on_sc=False),
)(_sc_gather_body)


BLK = 2048


def _mlp_body(x_ref, w1_ref, b1_ref, w2_ref, b2_ref, w3_ref, b3_ref, o_ref):
    h = jnp.dot(w1_ref[...], x_ref[...], preferred_element_type=jnp.float32)
    h = jnp.maximum(h + b1_ref[...], 0.0)
    h = jnp.dot(w2_ref[...], h, preferred_element_type=jnp.float32)
    h = jnp.maximum(h + b2_ref[...], 0.0)
    y = jnp.dot(w3_ref[...], h, preferred_element_type=jnp.float32)
    o_ref[...] = y[0] + b3_ref[0, 0]


def _mlp(xT, W1, b1c, W2, b2c, W3, b3c):
    grid = (B // BLK,)
    full = lambda shape: pl.BlockSpec(shape, lambda i: (0, 0))
    return pl.pallas_call(
        _mlp_body,
        grid=grid,
        in_specs=[
            pl.BlockSpec((F, BLK), lambda i: (0, i)),
            full((F, F)),
            full((F, 1)),
            full((D, F)),
            full((D, 1)),
            full((1, D)),
            full((1, 1)),
        ],
        out_specs=pl.BlockSpec((BLK,), lambda i: (i,)),
        out_shape=jax.ShapeDtypeStruct((B,), jnp.float32),
    )(xT, W1, b1c, W2, b2c, W3, b3c)


def kernel(user_idx, item_idx, user_table, item_table, W1, b1, W2, b2, W3, b3):
    ui = user_idx.astype(jnp.int32)
    ii = item_idx.astype(jnp.int32)
    flat = jnp.concatenate([user_table.reshape(NV), item_table.reshape(NV)])
    feat = jnp.arange(D, dtype=jnp.int32)[:, None]
    gu = ui[None, :] * D + feat
    gi = ii[None, :] * D + feat + NV
    g = jnp.concatenate([gu, gi], axis=0)
    g3 = g.reshape(F, NW, BPW).transpose(1, 0, 2).reshape(-1)
    xT = _sc_gather(g3, flat)
    out = _mlp(xT, W1, b1.reshape(F, 1), W2, b2.reshape(D, 1),
               W3, b3.reshape(1, 1))
    return out
